# baseline (speedup 1.0000x reference)
import functools

import jax
import jax.numpy as jnp
from jax import lax
from jax.experimental import pallas as pl
from jax.experimental.pallas import tpu as pltpu
from jax.experimental.pallas import tpu_sc as plsc

N_NODES = 10000
N_EDGES = 320000
D = 128

NC = 2
NS = 16
B = 128
BLK = 8
MCHT = 160
MC0 = 112
MC1 = MCHT - MC0
NBLK0 = MC0 // BLK
NBLK1 = MC1 // BLK
E_PAD = NS * MCHT * B
N_PAD = 10240
RPS = N_PAD // NS
ZCH = 128
CW = 8


def _sc_agg_body(with_count, *refs):
    if with_count:
        (x_hbm, src_hbm, dst_hbm, z128_hbm, z8_hbm, ones_hbm,
         part_hbm, cntp_hbm,
         agg_sh, cnt_sh, srcb, dstb, rows_v, ones_v,
         sg0, sg1, ss0, ss1, si0, si1) = refs
    else:
        (x_hbm, src_hbm, dst_hbm, z128_hbm,
         part_hbm,
         agg_sh, srcb, dstb, rows_v, ones_v,
         sg0, sg1, ss0, ss1, si0, si1) = refs
    sg = (sg0, sg1)
    ss = (ss0, ss1)
    si = (si0, si1)

    c = lax.axis_index("c")
    s = lax.axis_index("s")
    cbase = lax.select(c == 0, 0, MC0)
    nblk = lax.select(c == 0, NBLK0, NBLK1)

    pltpu.sync_copy(z128_hbm, rows_v.at[0])
    pltpu.sync_copy(z128_hbm, rows_v.at[1])
    for k in range(RPS // ZCH):
        pltpu.sync_copy(rows_v.at[0], agg_sh.at[pl.ds(s * RPS + k * ZCH, ZCH)])
    if with_count:
        pltpu.sync_copy(z8_hbm, ones_v)
        for k in range(RPS // ZCH):
            pltpu.sync_copy(ones_v, cnt_sh.at[pl.ds(s * RPS + k * ZCH, ZCH)])
        pltpu.sync_copy(ones_hbm, ones_v)

    pltpu.sync_copy(src_hbm.at[s, pl.ds(cbase, BLK)], srcb.at[0])
    pltpu.sync_copy(dst_hbm.at[s, pl.ds(cbase, BLK)], dstb.at[0])

    plsc.subcore_barrier()

    pltpu.async_copy(x_hbm.at[srcb.at[0, 0]], rows_v.at[0], sg[0])
    pltpu.async_copy(rows_v.at[1], agg_sh.at[dstb.at[0, 0]], ss[1], add=True)

    def block(t, carry):
        tb = lax.rem(t, 2)
        tbn = lax.rem(t + 1, 2)
        not_last = t < nblk - 1
        for k in range(BLK):
            bi = k % 2
            bo = 1 - bi
            pltpu.make_async_copy(
                rows_v.at[bo], agg_sh.at[dstb.at[tb, k]], ss[bo]).wait()
            if k < BLK - 1:
                pltpu.async_copy(
                    x_hbm.at[srcb.at[tb, k + 1]], rows_v.at[bo], sg[bo])
            else:
                @pl.when(not_last)
                def _():
                    pltpu.make_async_copy(
                        src_hbm.at[s, pl.ds(cbase + (t + 1) * BLK, BLK)],
                        srcb.at[tbn], si[0]).wait()
                    pltpu.make_async_copy(
                        dst_hbm.at[s, pl.ds(cbase + (t + 1) * BLK, BLK)],
                        dstb.at[tbn], si[0]).wait()
                    pltpu.async_copy(
                        x_hbm.at[srcb.at[tbn, 0]], rows_v.at[bo], sg[bo])
            pltpu.make_async_copy(
                x_hbm.at[srcb.at[tb, k]], rows_v.at[bi], sg[bi]).wait()
            pltpu.async_copy(
                rows_v.at[bi], agg_sh.at[dstb.at[tb, k]], ss[bi], add=True)
            if with_count:
                pltpu.sync_copy(ones_v, cnt_sh.at[dstb.at[tb, k]], add=True)
            if k == 1:
                @pl.when(not_last)
                def _():
                    pltpu.async_copy(
                        src_hbm.at[s, pl.ds(cbase + (t + 1) * BLK, BLK)],
                        srcb.at[tbn], si[0])
                    pltpu.async_copy(
                        dst_hbm.at[s, pl.ds(cbase + (t + 1) * BLK, BLK)],
                        dstb.at[tbn], si[0])
        return carry

    lax.fori_loop(0, nblk, block, 0)
    pltpu.make_async_copy(
        rows_v.at[1], agg_sh.at[dstb.at[0, 0]], ss[1]).wait()

    plsc.subcore_barrier()

    for k in range(RPS // ZCH):
        r0 = s * RPS + k * ZCH
        pltpu.sync_copy(agg_sh.at[pl.ds(r0, ZCH)], rows_v.at[0])
        pltpu.sync_copy(rows_v.at[0], part_hbm.at[c, pl.ds(r0, ZCH)])
        if with_count:
            pltpu.sync_copy(cnt_sh.at[pl.ds(r0, ZCH)], ones_v)
            pltpu.sync_copy(ones_v, cntp_hbm.at[c, pl.ds(r0, ZCH)])


def _make_sc_agg(with_count):
    mesh = plsc.VectorSubcoreMesh(
        core_axis_name="c", subcore_axis_name="s",
        num_cores=NC, num_subcores=NS)
    sems = [pltpu.SemaphoreType.DMA] * 6
    if with_count:
        out_type = (
            jax.ShapeDtypeStruct((NC, N_PAD, D), jnp.float32),
            jax.ShapeDtypeStruct((NC, N_PAD, CW), jnp.float32),
        )
        scratch = [
            pltpu.VMEM_SHARED((N_PAD, D), jnp.float32),
            pltpu.VMEM_SHARED((N_PAD, CW), jnp.float32),
            pltpu.VMEM((2, BLK, B), jnp.int32),
            pltpu.VMEM((2, BLK, B), jnp.int32),
            pltpu.VMEM((2, ZCH, D), jnp.float32),
            pltpu.VMEM((B, CW), jnp.float32),
        ] + sems
    else:
        out_type = jax.ShapeDtypeStruct((NC, N_PAD, D), jnp.float32)
        scratch = [
            pltpu.VMEM_SHARED((N_PAD, D), jnp.float32),
            pltpu.VMEM((2, BLK, B), jnp.int32),
            pltpu.VMEM((2, BLK, B), jnp.int32),
            pltpu.VMEM((2, ZCH, D), jnp.float32),
            pltpu.VMEM((B, CW), jnp.float32),
        ] + sems
    return pl.kernel(
        functools.partial(_sc_agg_body, with_count),
        out_type=out_type, mesh=mesh, scratch_types=scratch,
        compiler_params=pltpu.CompilerParams(use_tc_tiling_on_sc=False),
        name=f"sage_sc_agg_cnt{int(with_count)}")


_R = 1000


def _tc_dense_body(p0, p1, c0, c1, x, wl, wr, b, o):
    agg = p0[...] + p1[...]
    cnt = c0[:, 0:1] + c1[:, 0:1]
    inv = 1.0 / jnp.maximum(cnt, 1.0)
    g = jnp.dot(agg, wl[...], preferred_element_type=jnp.float32)
    h = jnp.dot(x[...], wr[...], preferred_element_type=jnp.float32)
    o[...] = jnp.maximum(g * inv + h + b[...], 0.0)


def _tc_dense(part, cntp, x, wl, wr, b):
    grid = (N_NODES // _R,)
    row = pl.BlockSpec((_R, D), lambda i: (i, 0))
    cb = pl.BlockSpec((_R, CW), lambda i: (i, 0))
    full = pl.BlockSpec((D, D), lambda i: (0, 0))
    bias = pl.BlockSpec((1, D), lambda i: (0, 0))
    return pl.pallas_call(
        _tc_dense_body,
        grid=grid,
        in_specs=[row, row, cb, cb, row, full, full, bias],
        out_specs=row,
        out_shape=jax.ShapeDtypeStruct((N_NODES, D), jnp.float32),
    )(part[0], part[1], cntp[0], cntp[1], x, wl, wr, b.reshape(1, D))


def kernel(x, edge_index, Wl1, Wr1, b1, Wl2, Wr2, b2):
    n_extra = E_PAD - N_EDGES
    src = edge_index[0].astype(jnp.int32)
    dst = edge_index[1].astype(jnp.int32)
    src = jnp.concatenate([src, jnp.zeros((n_extra,), jnp.int32)])
    pad_dst = N_NODES + (jnp.arange(n_extra, dtype=jnp.int32) % (N_PAD - N_NODES))
    dst = jnp.concatenate([dst, pad_dst])
    src = src.reshape(NS, MCHT, B)
    dst = dst.reshape(NS, MCHT, B)
    x = x.astype(jnp.float32)
    z128 = jnp.zeros((ZCH, D), jnp.float32)
    z8 = jnp.zeros((ZCH, CW), jnp.float32)
    ones = jnp.ones((B, CW), jnp.float32)

    part1, cntp = _make_sc_agg(True)(x, src, dst, z128, z8, ones)
    h = _tc_dense(part1, cntp, x, Wl1, Wr1, b1)
    part2 = _make_sc_agg(False)(h, src, dst, z128)
    out = _tc_dense(part2, cntp, h, Wl2, Wr2, b2)
    return out
